# baseline (speedup 1.0000x reference)
import functools

import jax
import jax.numpy as jnp
from jax import lax
from jax.experimental import pallas as pl
from jax.experimental.pallas import tpu as pltpu
from jax.experimental.pallas import tpu_sc as plsc

N_NODES = 10000
N_EDGES = 320000
D_IN = 128
D_HID = 128
D_OUT = 64

NC, NS = 2, 16
N_PAD = 10240
E_ROWS = 2560
E_PAD = E_ROWS * 128
R_CORE0 = 160
R_CORE1 = E_ROWS // NS - R_CORE0
GROUP = 16
SUB_ROWS = N_PAD // NS
DEG_W = 128

def _sc_mesh():
    return plsc.VectorSubcoreMesh(core_axis_name="c", subcore_axis_name="s")


NBUF = 2


def _sc_scatter(D):
    @functools.partial(
        pl.kernel,
        out_type=jax.ShapeDtypeStruct((NC, N_PAD, D), jnp.float32),
        mesh=_sc_mesh(),
        scratch_types=[
            pltpu.VMEM((GROUP, 128), jnp.int32),
            pltpu.VMEM((GROUP, 128), jnp.int32),
        ] + [pltpu.VMEM((128, D), jnp.float32)] * NBUF
          + [pltpu.SemaphoreType.DMA] * (2 * NBUF)
          + [pltpu.VMEM_SHARED((N_PAD, D), jnp.float32)],
    )
    def k(xs_hbm, src_hbm, dst_hbm, zeros_hbm, out_hbm, src_v, dst_v, *rest):
        bufs = rest[:NBUF]
        gsem = rest[NBUF:2 * NBUF]
        ssem = rest[2 * NBUF:3 * NBUF]
        acc_sh = rest[3 * NBUF]
        c = lax.axis_index("c")
        s = lax.axis_index("s")
        base = lax.select(c == 0, s * R_CORE0, NS * R_CORE0 + s * R_CORE1)
        ngroups = lax.select(
            c == 0, R_CORE0 // GROUP, R_CORE1 // GROUP)
        pltpu.sync_copy(zeros_hbm, acc_sh.at[pl.ds(s * SUB_ROWS, SUB_ROWS)])
        plsc.subcore_barrier()

        NSPLIT = 4

        class _Gather:
            def __init__(self, r, i):
                step = 128 // NSPLIT
                self.cps = [
                    pltpu.make_async_copy(
                        xs_hbm.at[src_v.at[r, pl.ds(p * step, step)]],
                        bufs[i].at[pl.ds(p * step, step)],
                        gsem[i],
                    )
                    for p in range(NSPLIT)
                ]

            def start(self):
                for cp in self.cps:
                    cp.start()

            def wait(self):
                for cp in self.cps:
                    cp.wait()

        gather = _Gather

        def scatter_cp(r, i):
            return pltpu.make_async_copy(
                bufs[i], acc_sh.at[dst_v.at[r]], ssem[i])

        @pl.loop(0, ngroups)
        def _(g):
            row0 = base + g * GROUP
            pltpu.sync_copy(src_hbm.at[pl.ds(row0, GROUP)], src_v)
            pltpu.sync_copy(dst_hbm.at[pl.ds(row0, GROUP)], dst_v)
            gather(0, 0).start()
            gather(1, 1).start()
            gather(0, 0).wait()
            scatter_cp(0, 0).start(add=True)
            for r in range(1, GROUP):
                i, o = r % 2, (r + 1) % 2
                gather(r, i).wait()
                scatter_cp(r - 1, o).wait()
                if r + 1 < GROUP:
                    gather(r + 1, o).start()
                scatter_cp(r, i).start(add=True)
            scatter_cp(GROUP - 1, (GROUP - 1) % 2).wait()

        plsc.subcore_barrier()
        pltpu.sync_copy(
            acc_sh.at[pl.ds(s * SUB_ROWS, SUB_ROWS)],
            out_hbm.at[c].at[pl.ds(s * SUB_ROWS, SUB_ROWS)],
        )

    return k


_R = 2048


def _dinv_of(degp_ref):
    deg = 1.0 + degp_ref[0, :, 0:1] + degp_ref[1, :, 0:1]
    return lax.rsqrt(deg)


def _t_matmul(x, w):
    def body(x_ref, w_ref, o_ref):
        o_ref[...] = jnp.dot(x_ref[...], w_ref[...],
                             preferred_element_type=jnp.float32)

    n, d = x.shape
    dout = w.shape[1]
    return pl.pallas_call(
        body,
        grid=(n // _R,),
        in_specs=[
            pl.BlockSpec((_R, d), lambda i: (i, 0)),
            pl.BlockSpec((d, dout), lambda i: (0, 0)),
        ],
        out_specs=pl.BlockSpec((_R, dout), lambda i: (i, 0)),
        out_shape=jax.ShapeDtypeStruct((n, dout), jnp.float32),
    )(x, w)


def _t_scale(xw, degp):
    def body(xw_ref, degp_ref, o_ref):
        o_ref[...] = xw_ref[...] * _dinv_of(degp_ref)

    n, d = xw.shape
    return pl.pallas_call(
        body,
        grid=(n // _R,),
        in_specs=[
            pl.BlockSpec((_R, d), lambda i: (i, 0)),
            pl.BlockSpec((2, _R, DEG_W), lambda i: (0, i, 0)),
        ],
        out_specs=pl.BlockSpec((_R, d), lambda i: (i, 0)),
        out_shape=jax.ShapeDtypeStruct((n, d), jnp.float32),
    )(xw, degp)


def _t_combine_matmul(s, xs, degp, b, w, relu):
    n, d = xs.shape
    dout = w.shape[1]
    opad = max(dout, 128)

    def body(s_ref, xs_ref, degp_ref, b_ref, w_ref, o_ref):
        dinv = _dinv_of(degp_ref)
        h = dinv * (s_ref[0] + s_ref[1] + xs_ref[...]) + b_ref[...]
        if relu:
            h = jnp.maximum(h, 0.0)
        r = jnp.dot(h, w_ref[...], preferred_element_type=jnp.float32) * dinv
        if opad != dout:
            r = jnp.concatenate(
                [r, jnp.zeros((r.shape[0], opad - dout), jnp.float32)], axis=1)
        o_ref[...] = r

    return pl.pallas_call(
        body,
        grid=(n // _R,),
        in_specs=[
            pl.BlockSpec((2, _R, d), lambda i: (0, i, 0)),
            pl.BlockSpec((_R, d), lambda i: (i, 0)),
            pl.BlockSpec((2, _R, DEG_W), lambda i: (0, i, 0)),
            pl.BlockSpec((1, d), lambda i: (0, 0)),
            pl.BlockSpec((d, dout), lambda i: (0, 0)),
        ],
        out_specs=pl.BlockSpec((_R, opad), lambda i: (i, 0)),
        out_shape=jax.ShapeDtypeStruct((n, opad), jnp.float32),
    )(s, xs, degp, b, w)


def _t_final(s, xs, degp, b, dout):
    n, d = xs.shape

    def body(s_ref, xs_ref, degp_ref, b_ref, o_ref):
        dinv = _dinv_of(degp_ref)
        o = dinv * (s_ref[0][:, :dout] + s_ref[1][:, :dout]
                    + xs_ref[:, :dout]) + b_ref[...]
        m = jnp.max(o, axis=1, keepdims=True)
        lse = jnp.log(jnp.sum(jnp.exp(o - m), axis=1, keepdims=True)) + m
        o_ref[...] = o - lse

    return pl.pallas_call(
        body,
        grid=(n // _R,),
        in_specs=[
            pl.BlockSpec((2, _R, d), lambda i: (0, i, 0)),
            pl.BlockSpec((_R, d), lambda i: (i, 0)),
            pl.BlockSpec((2, _R, DEG_W), lambda i: (0, i, 0)),
            pl.BlockSpec((1, dout), lambda i: (0, 0)),
        ],
        out_specs=pl.BlockSpec((_R, dout), lambda i: (i, 0)),
        out_shape=jax.ShapeDtypeStruct((n, dout), jnp.float32),
    )(s, xs, degp, b)


def kernel(X, edge_index, W1, b1, W2, b2):
    ei = edge_index.astype(jnp.int32)
    pad = jnp.full((E_PAD - N_EDGES,), N_NODES, jnp.int32)
    src = jnp.concatenate([ei[0], pad]).reshape(E_ROWS, 128)
    dst = jnp.concatenate([ei[1], pad]).reshape(E_ROWS, 128)
    Xp = jnp.pad(X, ((0, N_PAD - N_NODES), (0, 0)))

    zeros_hid = jnp.zeros((SUB_ROWS, D_HID), jnp.float32)

    ones_tbl = jnp.ones((N_PAD, D_HID), jnp.float32)
    degp = _sc_scatter(D_HID)(ones_tbl, dst, dst, zeros_hid)
    xw1 = _t_matmul(Xp, W1)
    xs1 = _t_scale(xw1, degp)

    s1 = _sc_scatter(D_HID)(xs1, src, dst, zeros_hid)
    xs2 = _t_combine_matmul(s1, xs1, degp, b1.reshape(1, -1), W1, relu=False)

    s2 = _sc_scatter(D_HID)(xs2, src, dst, zeros_hid)
    xs3 = _t_combine_matmul(s2, xs2, degp, b1.reshape(1, -1), W2, relu=True)

    s3 = _sc_scatter(D_HID)(xs3, src, dst, zeros_hid)
    y = _t_final(s3, xs3, degp, b2.reshape(1, -1), D_OUT)
    return y[:N_NODES]


# speedup vs baseline: 1.3641x; 1.3641x over previous
import functools

import jax
import jax.numpy as jnp
from jax import lax
from jax.experimental import pallas as pl
from jax.experimental.pallas import tpu as pltpu
from jax.experimental.pallas import tpu_sc as plsc

N_NODES = 10000
N_EDGES = 320000
D_IN = 128
D_HID = 128
D_OUT = 64

NC, NS = 2, 16
N_PAD = 10240
E_ROWS = 2560
E_PAD = E_ROWS * 128
R_CORE0 = 128
R_CORE1 = E_ROWS // NS - R_CORE0
GROUP = 16
SUB_ROWS = N_PAD // NS
DEG_W = 128

def _sc_mesh():
    return plsc.VectorSubcoreMesh(core_axis_name="c", subcore_axis_name="s")


NBUF = 2


def _sc_scatter(D):
    @functools.partial(
        pl.kernel,
        out_type=jax.ShapeDtypeStruct((NC, N_PAD, D), jnp.float32),
        mesh=_sc_mesh(),
        scratch_types=[
            pltpu.VMEM((GROUP, 128), jnp.int32),
            pltpu.VMEM((GROUP, 128), jnp.int32),
        ] + [pltpu.VMEM((128, D), jnp.float32)] * NBUF
          + [pltpu.SemaphoreType.DMA] * (2 * NBUF)
          + [pltpu.VMEM_SHARED((N_PAD, D), jnp.float32)],
    )
    def k(xs_hbm, src_hbm, dst_hbm, zeros_hbm, out_hbm, src_v, dst_v, *rest):
        bufs = rest[:NBUF]
        gsem = rest[NBUF:2 * NBUF]
        ssem = rest[2 * NBUF:3 * NBUF]
        acc_sh = rest[3 * NBUF]
        c = lax.axis_index("c")
        s = lax.axis_index("s")
        base = lax.select(c == 0, s * R_CORE0, NS * R_CORE0 + s * R_CORE1)
        ngroups = lax.select(
            c == 0, R_CORE0 // GROUP, R_CORE1 // GROUP)
        pltpu.sync_copy(zeros_hbm, acc_sh.at[pl.ds(s * SUB_ROWS, SUB_ROWS)])
        plsc.subcore_barrier()

        NSPLIT = 4

        class _Gather:
            def __init__(self, r, i):
                step = 128 // NSPLIT
                self.cps = [
                    pltpu.make_async_copy(
                        xs_hbm.at[src_v.at[r, pl.ds(p * step, step)]],
                        bufs[i].at[pl.ds(p * step, step)],
                        gsem[i],
                    )
                    for p in range(NSPLIT)
                ]

            def start(self):
                for cp in self.cps:
                    cp.start()

            def wait(self):
                for cp in self.cps:
                    cp.wait()

        gather = _Gather

        def scatter_cp(r, i):
            return pltpu.make_async_copy(
                bufs[i], acc_sh.at[dst_v.at[r]], ssem[i])

        @pl.loop(0, ngroups)
        def _(g):
            row0 = base + g * GROUP
            pltpu.sync_copy(src_hbm.at[pl.ds(row0, GROUP)], src_v)
            pltpu.sync_copy(dst_hbm.at[pl.ds(row0, GROUP)], dst_v)
            gather(0, 0).start()
            gather(1, 1).start()
            gather(0, 0).wait()
            scatter_cp(0, 0).start(add=True)
            for r in range(1, GROUP):
                i, o = r % 2, (r + 1) % 2
                gather(r, i).wait()
                scatter_cp(r - 1, o).wait()
                if r + 1 < GROUP:
                    gather(r + 1, o).start()
                scatter_cp(r, i).start(add=True)
            scatter_cp(GROUP - 1, (GROUP - 1) % 2).wait()

        plsc.subcore_barrier()
        pltpu.sync_copy(
            acc_sh.at[pl.ds(s * SUB_ROWS, SUB_ROWS)],
            out_hbm.at[c].at[pl.ds(s * SUB_ROWS, SUB_ROWS)],
        )

    return k


_R = 2048


def _dinv_of(degp_ref):
    deg = 1.0 + degp_ref[0, :, 0:1] + degp_ref[1, :, 0:1]
    return lax.rsqrt(deg)


def _t_matmul(x, w):
    def body(x_ref, w_ref, o_ref):
        o_ref[...] = jnp.dot(x_ref[...], w_ref[...],
                             preferred_element_type=jnp.float32)

    n, d = x.shape
    dout = w.shape[1]
    return pl.pallas_call(
        body,
        grid=(n // _R,),
        in_specs=[
            pl.BlockSpec((_R, d), lambda i: (i, 0)),
            pl.BlockSpec((d, dout), lambda i: (0, 0)),
        ],
        out_specs=pl.BlockSpec((_R, dout), lambda i: (i, 0)),
        out_shape=jax.ShapeDtypeStruct((n, dout), jnp.float32),
    )(x, w)


def _t_scale(xw, degp):
    def body(xw_ref, degp_ref, o_ref):
        o_ref[...] = xw_ref[...] * _dinv_of(degp_ref)

    n, d = xw.shape
    return pl.pallas_call(
        body,
        grid=(n // _R,),
        in_specs=[
            pl.BlockSpec((_R, d), lambda i: (i, 0)),
            pl.BlockSpec((2, _R, DEG_W), lambda i: (0, i, 0)),
        ],
        out_specs=pl.BlockSpec((_R, d), lambda i: (i, 0)),
        out_shape=jax.ShapeDtypeStruct((n, d), jnp.float32),
    )(xw, degp)


def _t_combine_matmul(s, xs, degp, b, w, relu):
    n, d = xs.shape
    dout = w.shape[1]
    opad = max(dout, 128)

    def body(s_ref, xs_ref, degp_ref, b_ref, w_ref, o_ref):
        dinv = _dinv_of(degp_ref)
        h = dinv * (s_ref[0] + s_ref[1] + xs_ref[...]) + b_ref[...]
        if relu:
            h = jnp.maximum(h, 0.0)
        r = jnp.dot(h, w_ref[...], preferred_element_type=jnp.float32) * dinv
        if opad != dout:
            r = jnp.concatenate(
                [r, jnp.zeros((r.shape[0], opad - dout), jnp.float32)], axis=1)
        o_ref[...] = r

    return pl.pallas_call(
        body,
        grid=(n // _R,),
        in_specs=[
            pl.BlockSpec((2, _R, d), lambda i: (0, i, 0)),
            pl.BlockSpec((_R, d), lambda i: (i, 0)),
            pl.BlockSpec((2, _R, DEG_W), lambda i: (0, i, 0)),
            pl.BlockSpec((1, d), lambda i: (0, 0)),
            pl.BlockSpec((d, dout), lambda i: (0, 0)),
        ],
        out_specs=pl.BlockSpec((_R, opad), lambda i: (i, 0)),
        out_shape=jax.ShapeDtypeStruct((n, opad), jnp.float32),
    )(s, xs, degp, b, w)


def _t_final(s, xs, degp, b, dout):
    n, d = xs.shape

    def body(s_ref, xs_ref, degp_ref, b_ref, o_ref):
        dinv = _dinv_of(degp_ref)
        o = dinv * (s_ref[0][:, :dout] + s_ref[1][:, :dout]
                    + xs_ref[:, :dout]) + b_ref[...]
        m = jnp.max(o, axis=1, keepdims=True)
        lse = jnp.log(jnp.sum(jnp.exp(o - m), axis=1, keepdims=True)) + m
        o_ref[...] = o - lse

    return pl.pallas_call(
        body,
        grid=(n // _R,),
        in_specs=[
            pl.BlockSpec((2, _R, d), lambda i: (0, i, 0)),
            pl.BlockSpec((_R, d), lambda i: (i, 0)),
            pl.BlockSpec((2, _R, DEG_W), lambda i: (0, i, 0)),
            pl.BlockSpec((1, dout), lambda i: (0, 0)),
        ],
        out_specs=pl.BlockSpec((_R, dout), lambda i: (i, 0)),
        out_shape=jax.ShapeDtypeStruct((n, dout), jnp.float32),
    )(s, xs, degp, b)


def kernel(X, edge_index, W1, b1, W2, b2):
    ei = edge_index.astype(jnp.int32)
    pad = jnp.full((E_PAD - N_EDGES,), N_NODES, jnp.int32)
    src = jnp.concatenate([ei[0], pad]).reshape(E_ROWS, 128)
    dst = jnp.concatenate([ei[1], pad]).reshape(E_ROWS, 128)
    Xp = jnp.pad(X, ((0, N_PAD - N_NODES), (0, 0)))

    zeros_hid = jnp.zeros((SUB_ROWS, D_HID), jnp.float32)

    ones_tbl = jnp.ones((N_PAD, D_HID), jnp.float32)
    degp = _sc_scatter(D_HID)(ones_tbl, dst, dst, zeros_hid)
    xw1 = _t_matmul(Xp, W1)
    xs1 = _t_scale(xw1, degp)

    s1 = _sc_scatter(D_HID)(xs1, src, dst, zeros_hid)
    xs2 = _t_combine_matmul(s1, xs1, degp, b1.reshape(1, -1), W1, relu=False)

    s2 = _sc_scatter(D_HID)(xs2, src, dst, zeros_hid)
    xs3 = _t_combine_matmul(s2, xs2, degp, b1.reshape(1, -1), W2, relu=True)

    s3 = _sc_scatter(D_HID)(xs3, src, dst, zeros_hid)
    y = _t_final(s3, xs3, degp, b2.reshape(1, -1), D_OUT)
    return y[:N_NODES]


# speedup vs baseline: 1.5124x; 1.1087x over previous
import functools

import jax
import jax.numpy as jnp
from jax import lax
from jax.experimental import pallas as pl
from jax.experimental.pallas import tpu as pltpu
from jax.experimental.pallas import tpu_sc as plsc

N_NODES = 10000
N_EDGES = 320000
D_IN = 128
D_HID = 128
D_OUT = 64

NC, NS = 2, 16
N_PAD = 10240
E_ROWS = 2560
E_PAD = E_ROWS * 128
R_CORE0 = 144
R_CORE1 = E_ROWS // NS - R_CORE0
GROUP = 16
SUB_ROWS = N_PAD // NS
DEG_W = 128

def _sc_mesh():
    return plsc.VectorSubcoreMesh(core_axis_name="c", subcore_axis_name="s")


NBUF = 2


def _sc_scatter(D):
    @functools.partial(
        pl.kernel,
        out_type=jax.ShapeDtypeStruct((NC, N_PAD, D), jnp.float32),
        mesh=_sc_mesh(),
        scratch_types=[
            pltpu.VMEM((GROUP, 128), jnp.int32),
            pltpu.VMEM((GROUP, 128), jnp.int32),
        ] + [pltpu.VMEM((128, D), jnp.float32)] * NBUF
          + [pltpu.SemaphoreType.DMA] * (2 * NBUF)
          + [pltpu.VMEM_SHARED((N_PAD, D), jnp.float32)],
    )
    def k(xs_hbm, src_hbm, dst_hbm, zeros_hbm, out_hbm, src_v, dst_v, *rest):
        bufs = rest[:NBUF]
        gsem = rest[NBUF:2 * NBUF]
        ssem = rest[2 * NBUF:3 * NBUF]
        acc_sh = rest[3 * NBUF]
        c = lax.axis_index("c")
        s = lax.axis_index("s")
        base = lax.select(c == 0, s * R_CORE0, NS * R_CORE0 + s * R_CORE1)
        ngroups = lax.select(
            c == 0, R_CORE0 // GROUP, R_CORE1 // GROUP)
        pltpu.sync_copy(zeros_hbm, acc_sh.at[pl.ds(s * SUB_ROWS, SUB_ROWS)])
        plsc.subcore_barrier()

        NSPLIT = 4

        class _Gather:
            def __init__(self, r, i):
                step = 128 // NSPLIT
                self.cps = [
                    pltpu.make_async_copy(
                        xs_hbm.at[src_v.at[r, pl.ds(p * step, step)]],
                        bufs[i].at[pl.ds(p * step, step)],
                        gsem[i],
                    )
                    for p in range(NSPLIT)
                ]

            def start(self):
                for cp in self.cps:
                    cp.start()

            def wait(self):
                for cp in self.cps:
                    cp.wait()

        gather = _Gather

        def scatter_cp(r, i):
            return pltpu.make_async_copy(
                bufs[i], acc_sh.at[dst_v.at[r]], ssem[i])

        @pl.loop(0, ngroups)
        def _(g):
            row0 = base + g * GROUP
            pltpu.sync_copy(src_hbm.at[pl.ds(row0, GROUP)], src_v)
            pltpu.sync_copy(dst_hbm.at[pl.ds(row0, GROUP)], dst_v)
            gather(0, 0).start()
            gather(1, 1).start()
            gather(0, 0).wait()
            scatter_cp(0, 0).start(add=True)
            for r in range(1, GROUP):
                i, o = r % 2, (r + 1) % 2
                gather(r, i).wait()
                scatter_cp(r - 1, o).wait()
                if r + 1 < GROUP:
                    gather(r + 1, o).start()
                scatter_cp(r, i).start(add=True)
            scatter_cp(GROUP - 1, (GROUP - 1) % 2).wait()

        plsc.subcore_barrier()
        pltpu.sync_copy(
            acc_sh.at[pl.ds(s * SUB_ROWS, SUB_ROWS)],
            out_hbm.at[c].at[pl.ds(s * SUB_ROWS, SUB_ROWS)],
        )

    return k


_R = 2048


def _dinv_of(degp_ref):
    deg = 1.0 + degp_ref[0, :, 0:1] + degp_ref[1, :, 0:1]
    return lax.rsqrt(deg)


def _t_matmul(x, w):
    def body(x_ref, w_ref, o_ref):
        o_ref[...] = jnp.dot(x_ref[...], w_ref[...],
                             preferred_element_type=jnp.float32)

    n, d = x.shape
    dout = w.shape[1]
    return pl.pallas_call(
        body,
        grid=(n // _R,),
        in_specs=[
            pl.BlockSpec((_R, d), lambda i: (i, 0)),
            pl.BlockSpec((d, dout), lambda i: (0, 0)),
        ],
        out_specs=pl.BlockSpec((_R, dout), lambda i: (i, 0)),
        out_shape=jax.ShapeDtypeStruct((n, dout), jnp.float32),
    )(x, w)


def _t_scale(xw, degp):
    def body(xw_ref, degp_ref, o_ref):
        o_ref[...] = xw_ref[...] * _dinv_of(degp_ref)

    n, d = xw.shape
    return pl.pallas_call(
        body,
        grid=(n // _R,),
        in_specs=[
            pl.BlockSpec((_R, d), lambda i: (i, 0)),
            pl.BlockSpec((2, _R, DEG_W), lambda i: (0, i, 0)),
        ],
        out_specs=pl.BlockSpec((_R, d), lambda i: (i, 0)),
        out_shape=jax.ShapeDtypeStruct((n, d), jnp.float32),
    )(xw, degp)


def _t_combine_matmul(s, xs, degp, b, w, relu):
    n, d = xs.shape
    dout = w.shape[1]
    opad = max(dout, 128)

    def body(s_ref, xs_ref, degp_ref, b_ref, w_ref, o_ref):
        dinv = _dinv_of(degp_ref)
        h = dinv * (s_ref[0] + s_ref[1] + xs_ref[...]) + b_ref[...]
        if relu:
            h = jnp.maximum(h, 0.0)
        r = jnp.dot(h, w_ref[...], preferred_element_type=jnp.float32) * dinv
        if opad != dout:
            r = jnp.concatenate(
                [r, jnp.zeros((r.shape[0], opad - dout), jnp.float32)], axis=1)
        o_ref[...] = r

    return pl.pallas_call(
        body,
        grid=(n // _R,),
        in_specs=[
            pl.BlockSpec((2, _R, d), lambda i: (0, i, 0)),
            pl.BlockSpec((_R, d), lambda i: (i, 0)),
            pl.BlockSpec((2, _R, DEG_W), lambda i: (0, i, 0)),
            pl.BlockSpec((1, d), lambda i: (0, 0)),
            pl.BlockSpec((d, dout), lambda i: (0, 0)),
        ],
        out_specs=pl.BlockSpec((_R, opad), lambda i: (i, 0)),
        out_shape=jax.ShapeDtypeStruct((n, opad), jnp.float32),
    )(s, xs, degp, b, w)


def _t_final(s, xs, degp, b, dout):
    n, d = xs.shape

    def body(s_ref, xs_ref, degp_ref, b_ref, o_ref):
        dinv = _dinv_of(degp_ref)
        o = dinv * (s_ref[0][:, :dout] + s_ref[1][:, :dout]
                    + xs_ref[:, :dout]) + b_ref[...]
        m = jnp.max(o, axis=1, keepdims=True)
        lse = jnp.log(jnp.sum(jnp.exp(o - m), axis=1, keepdims=True)) + m
        o_ref[...] = o - lse

    return pl.pallas_call(
        body,
        grid=(n // _R,),
        in_specs=[
            pl.BlockSpec((2, _R, d), lambda i: (0, i, 0)),
            pl.BlockSpec((_R, d), lambda i: (i, 0)),
            pl.BlockSpec((2, _R, DEG_W), lambda i: (0, i, 0)),
            pl.BlockSpec((1, dout), lambda i: (0, 0)),
        ],
        out_specs=pl.BlockSpec((_R, dout), lambda i: (i, 0)),
        out_shape=jax.ShapeDtypeStruct((n, dout), jnp.float32),
    )(s, xs, degp, b)


def kernel(X, edge_index, W1, b1, W2, b2):
    ei = edge_index.astype(jnp.int32)
    pad = jnp.full((E_PAD - N_EDGES,), N_NODES, jnp.int32)
    src = jnp.concatenate([ei[0], pad]).reshape(E_ROWS, 128)
    dst = jnp.concatenate([ei[1], pad]).reshape(E_ROWS, 128)
    Xp = jnp.pad(X, ((0, N_PAD - N_NODES), (0, 0)))

    zeros_hid = jnp.zeros((SUB_ROWS, D_HID), jnp.float32)

    ones_tbl = jnp.ones((N_PAD, D_HID), jnp.float32)
    degp = _sc_scatter(D_HID)(ones_tbl, dst, dst, zeros_hid)
    xw1 = _t_matmul(Xp, W1)
    xs1 = _t_scale(xw1, degp)

    s1 = _sc_scatter(D_HID)(xs1, src, dst, zeros_hid)
    xs2 = _t_combine_matmul(s1, xs1, degp, b1.reshape(1, -1), W1, relu=False)

    s2 = _sc_scatter(D_HID)(xs2, src, dst, zeros_hid)
    xs3 = _t_combine_matmul(s2, xs2, degp, b1.reshape(1, -1), W2, relu=True)

    s3 = _sc_scatter(D_HID)(xs3, src, dst, zeros_hid)
    y = _t_final(s3, xs3, degp, b2.reshape(1, -1), D_OUT)
    return y[:N_NODES]


# speedup vs baseline: 1.7344x; 1.1468x over previous
import functools

import jax
import jax.numpy as jnp
from jax import lax
from jax.experimental import pallas as pl
from jax.experimental.pallas import tpu as pltpu
from jax.experimental.pallas import tpu_sc as plsc

N_NODES = 10000
N_EDGES = 320000
D_IN = 128
D_HID = 128
D_OUT = 64

NC, NS = 2, 16
N_PAD = 10240
E_ROWS = 2560
E_PAD = E_ROWS * 128
R_CORE0 = 144
R_CORE1 = E_ROWS // NS - R_CORE0
GROUP = 16
SUB_ROWS = N_PAD // NS
DEG_W = 128

def _sc_mesh():
    return plsc.VectorSubcoreMesh(core_axis_name="c", subcore_axis_name="s")


NBUF = 2


def _sc_scatter(D, r_core0=R_CORE0):
    @functools.partial(
        pl.kernel,
        out_type=jax.ShapeDtypeStruct((NC, N_PAD, D), jnp.float32),
        mesh=_sc_mesh(),
        scratch_types=[
            pltpu.VMEM((GROUP, 128), jnp.int32),
            pltpu.VMEM((GROUP, 128), jnp.int32),
        ] + [pltpu.VMEM((128, D), jnp.float32)] * NBUF
          + [pltpu.SemaphoreType.DMA] * (2 * NBUF)
          + [pltpu.VMEM_SHARED((N_PAD, D), jnp.float32)],
    )
    def k(xs_hbm, src_hbm, dst_hbm, zeros_hbm, out_hbm, src_v, dst_v, *rest):
        bufs = rest[:NBUF]
        gsem = rest[NBUF:2 * NBUF]
        ssem = rest[2 * NBUF:3 * NBUF]
        acc_sh = rest[3 * NBUF]
        c = lax.axis_index("c")
        s = lax.axis_index("s")
        r_core1 = E_ROWS // NS - r_core0
        base = lax.select(c == 0, s * r_core0, NS * r_core0 + s * r_core1)
        ngroups = lax.select(c == 0, r_core0 // GROUP, r_core1 // GROUP)
        pltpu.sync_copy(zeros_hbm, acc_sh.at[pl.ds(s * SUB_ROWS, SUB_ROWS)])
        plsc.subcore_barrier()

        NSPLIT = 4

        class _Gather:
            def __init__(self, r, i):
                step = 128 // NSPLIT
                self.cps = [
                    pltpu.make_async_copy(
                        xs_hbm.at[src_v.at[r, pl.ds(p * step, step)]],
                        bufs[i].at[pl.ds(p * step, step)],
                        gsem[i],
                    )
                    for p in range(NSPLIT)
                ]

            def start(self):
                for cp in self.cps:
                    cp.start()

            def wait(self):
                for cp in self.cps:
                    cp.wait()

        gather = _Gather

        def scatter_cp(r, i):
            return pltpu.make_async_copy(
                bufs[i], acc_sh.at[dst_v.at[r]], ssem[i])

        @pl.loop(0, ngroups)
        def _(g):
            row0 = base + g * GROUP
            pltpu.sync_copy(src_hbm.at[pl.ds(row0, GROUP)], src_v)
            pltpu.sync_copy(dst_hbm.at[pl.ds(row0, GROUP)], dst_v)
            gather(0, 0).start()
            gather(1, 1).start()
            gather(0, 0).wait()
            scatter_cp(0, 0).start(add=True)
            for r in range(1, GROUP):
                i, o = r % 2, (r + 1) % 2
                gather(r, i).wait()
                scatter_cp(r - 1, o).wait()
                if r + 1 < GROUP:
                    gather(r + 1, o).start()
                scatter_cp(r, i).start(add=True)
            scatter_cp(GROUP - 1, (GROUP - 1) % 2).wait()

        plsc.subcore_barrier()
        pltpu.sync_copy(
            acc_sh.at[pl.ds(s * SUB_ROWS, SUB_ROWS)],
            out_hbm.at[c].at[pl.ds(s * SUB_ROWS, SUB_ROWS)],
        )

    return k


def _sc_degree():
    r_core0 = E_ROWS // NS // 2

    @functools.partial(
        pl.kernel,
        out_type=jax.ShapeDtypeStruct((NC, N_PAD, 128), jnp.float32),
        mesh=_sc_mesh(),
        scratch_types=[
            pltpu.VMEM((GROUP, 128), jnp.int32),
        ] + [pltpu.VMEM((128, 128), jnp.float32)] * NBUF
          + [pltpu.SemaphoreType.DMA] * (2 * NBUF)
          + [pltpu.VMEM_SHARED((128, 128), jnp.float32),
             pltpu.VMEM_SHARED((N_PAD, 128), jnp.float32)],
    )
    def k(dst_hbm, ones_hbm, zeros_hbm, out_hbm, dst_v, *rest):
        bufs = rest[:NBUF]
        gsem = rest[NBUF:2 * NBUF]
        ssem = rest[2 * NBUF:3 * NBUF]
        ones_sh = rest[3 * NBUF]
        acc_sh = rest[3 * NBUF + 1]
        c = lax.axis_index("c")
        s = lax.axis_index("s")
        r_core1 = E_ROWS // NS - r_core0
        base = lax.select(c == 0, s * r_core0, NS * r_core0 + s * r_core1)
        ngroups = lax.select(c == 0, r_core0 // GROUP, r_core1 // GROUP)
        pltpu.sync_copy(zeros_hbm, acc_sh.at[pl.ds(s * SUB_ROWS, SUB_ROWS)])

        @pl.when(s == 0)
        def _():
            pltpu.sync_copy(ones_hbm, ones_sh)

        plsc.subcore_barrier()

        def refresh(i):
            return pltpu.make_async_copy(ones_sh, bufs[i], gsem[i])

        def scatter_cp(r, i):
            return pltpu.make_async_copy(
                bufs[i], acc_sh.at[dst_v.at[r]], ssem[i])

        @pl.loop(0, ngroups)
        def _(g):
            row0 = base + g * GROUP
            pltpu.sync_copy(dst_hbm.at[pl.ds(row0, GROUP)], dst_v)
            refresh(0).start()
            refresh(1).start()
            refresh(0).wait()
            scatter_cp(0, 0).start(add=True)
            for r in range(1, GROUP):
                i, o = r % 2, (r + 1) % 2
                refresh(i).wait()
                scatter_cp(r - 1, o).wait()
                if r + 1 < GROUP:
                    refresh(o).start()
                scatter_cp(r, i).start(add=True)
            scatter_cp(GROUP - 1, (GROUP - 1) % 2).wait()

        plsc.subcore_barrier()
        pltpu.sync_copy(
            acc_sh.at[pl.ds(s * SUB_ROWS, SUB_ROWS)],
            out_hbm.at[c].at[pl.ds(s * SUB_ROWS, SUB_ROWS)],
        )

    return k


_R = 2048


def _dinv_of(degp_ref):
    deg = 1.0 + degp_ref[0, :, 0:1] + degp_ref[1, :, 0:1]
    return lax.rsqrt(deg)


def _t_matmul(x, w):
    def body(x_ref, w_ref, o_ref):
        o_ref[...] = jnp.dot(x_ref[...], w_ref[...],
                             preferred_element_type=jnp.float32)

    n, d = x.shape
    dout = w.shape[1]
    return pl.pallas_call(
        body,
        grid=(n // _R,),
        in_specs=[
            pl.BlockSpec((_R, d), lambda i: (i, 0)),
            pl.BlockSpec((d, dout), lambda i: (0, 0)),
        ],
        out_specs=pl.BlockSpec((_R, dout), lambda i: (i, 0)),
        out_shape=jax.ShapeDtypeStruct((n, dout), jnp.float32),
    )(x, w)


def _t_scale(xw, degp):
    def body(xw_ref, degp_ref, o_ref):
        o_ref[...] = xw_ref[...] * _dinv_of(degp_ref)

    n, d = xw.shape
    return pl.pallas_call(
        body,
        grid=(n // _R,),
        in_specs=[
            pl.BlockSpec((_R, d), lambda i: (i, 0)),
            pl.BlockSpec((2, _R, DEG_W), lambda i: (0, i, 0)),
        ],
        out_specs=pl.BlockSpec((_R, d), lambda i: (i, 0)),
        out_shape=jax.ShapeDtypeStruct((n, d), jnp.float32),
    )(xw, degp)


def _t_combine_matmul(s, xs, degp, b, w, relu):
    n, d = xs.shape
    dout = w.shape[1]
    opad = max(dout, 128)

    def body(s_ref, xs_ref, degp_ref, b_ref, w_ref, o_ref):
        dinv = _dinv_of(degp_ref)
        h = dinv * (s_ref[0] + s_ref[1] + xs_ref[...]) + b_ref[...]
        if relu:
            h = jnp.maximum(h, 0.0)
        r = jnp.dot(h, w_ref[...], preferred_element_type=jnp.float32) * dinv
        if opad != dout:
            r = jnp.concatenate(
                [r, jnp.zeros((r.shape[0], opad - dout), jnp.float32)], axis=1)
        o_ref[...] = r

    return pl.pallas_call(
        body,
        grid=(n // _R,),
        in_specs=[
            pl.BlockSpec((2, _R, d), lambda i: (0, i, 0)),
            pl.BlockSpec((_R, d), lambda i: (i, 0)),
            pl.BlockSpec((2, _R, DEG_W), lambda i: (0, i, 0)),
            pl.BlockSpec((1, d), lambda i: (0, 0)),
            pl.BlockSpec((d, dout), lambda i: (0, 0)),
        ],
        out_specs=pl.BlockSpec((_R, opad), lambda i: (i, 0)),
        out_shape=jax.ShapeDtypeStruct((n, opad), jnp.float32),
    )(s, xs, degp, b, w)


def _t_final(s, xs, degp, b, dout):
    n, d = xs.shape

    def body(s_ref, xs_ref, degp_ref, b_ref, o_ref):
        dinv = _dinv_of(degp_ref)
        o = dinv * (s_ref[0][:, :dout] + s_ref[1][:, :dout]
                    + xs_ref[:, :dout]) + b_ref[...]
        m = jnp.max(o, axis=1, keepdims=True)
        lse = jnp.log(jnp.sum(jnp.exp(o - m), axis=1, keepdims=True)) + m
        o_ref[...] = o - lse

    return pl.pallas_call(
        body,
        grid=(n // _R,),
        in_specs=[
            pl.BlockSpec((2, _R, d), lambda i: (0, i, 0)),
            pl.BlockSpec((_R, d), lambda i: (i, 0)),
            pl.BlockSpec((2, _R, DEG_W), lambda i: (0, i, 0)),
            pl.BlockSpec((1, dout), lambda i: (0, 0)),
        ],
        out_specs=pl.BlockSpec((_R, dout), lambda i: (i, 0)),
        out_shape=jax.ShapeDtypeStruct((n, dout), jnp.float32),
    )(s, xs, degp, b)


def kernel(X, edge_index, W1, b1, W2, b2):
    ei = edge_index.astype(jnp.int32)
    pad = jnp.full((E_PAD - N_EDGES,), N_NODES, jnp.int32)
    src = jnp.concatenate([ei[0], pad]).reshape(E_ROWS, 128)
    dst = jnp.concatenate([ei[1], pad]).reshape(E_ROWS, 128)
    Xp = jnp.pad(X, ((0, N_PAD - N_NODES), (0, 0)))

    zeros_hid = jnp.zeros((SUB_ROWS, D_HID), jnp.float32)

    ones_blk = jnp.ones((128, 128), jnp.float32)
    degp = _sc_degree()(dst, ones_blk, zeros_hid)
    xw1 = _t_matmul(Xp, W1)
    xs1 = _t_scale(xw1, degp)

    s1 = _sc_scatter(D_HID)(xs1, src, dst, zeros_hid)
    xs2 = _t_combine_matmul(s1, xs1, degp, b1.reshape(1, -1), W1, relu=False)

    s2 = _sc_scatter(D_HID)(xs2, src, dst, zeros_hid)
    xs3 = _t_combine_matmul(s2, xs2, degp, b1.reshape(1, -1), W2, relu=True)

    s3 = _sc_scatter(D_HID)(xs3, src, dst, zeros_hid)
    y = _t_final(s3, xs3, degp, b2.reshape(1, -1), D_OUT)
    return y[:N_NODES]
